# baseline (speedup 1.0000x reference)
import functools

import jax
import jax.numpy as jnp
from jax import lax
from jax.experimental import pallas as pl
from jax.experimental.pallas import tpu as pltpu
from jax.experimental.pallas import tpu_sc as plsc

NC = 2
NS = 16
L = 16
NW = NC * NS

BATCH = 16384
D = 128
BPW = BATCH // NW
CHUNK = 32
NCHUNK = BPW // CHUNK
NPAIR = NCHUNK // 2

_mesh = plsc.VectorSubcoreMesh(
    core_axis_name="c", subcore_axis_name="s", num_cores=NC, num_subcores=NS
)


@functools.partial(
    pl.kernel,
    out_type=jax.ShapeDtypeStruct((BATCH,), jnp.float32),
    mesh=_mesh,
    compiler_params=pltpu.CompilerParams(needs_layout_passes=False),
    scratch_types=[
        pltpu.VMEM((BPW,), jnp.int32),
        pltpu.VMEM((BPW,), jnp.int32),
        pltpu.VMEM((2, CHUNK, D), jnp.float32),
        pltpu.VMEM((2, CHUNK, D), jnp.float32),
        pltpu.VMEM((BPW,), jnp.float32),
        pltpu.VMEM((L, L + 1), jnp.float32),
        pltpu.SemaphoreType.DMA((2,)),
        pltpu.SemaphoreType.DMA((2,)),
    ],
)
def _mf_sc_kernel(
    uids_hbm, iids_hbm, utab_hbm, itab_hbm, out_hbm,
    uidx, iidx, ubuf, ibuf, outv, part,
    usem, isem,
):
    wid = lax.axis_index("s") * NC + lax.axis_index("c")
    base_row = wid * BPW
    hu = pltpu.async_copy(uids_hbm.at[pl.ds(base_row, BPW)], uidx, usem.at[0])
    hi = pltpu.async_copy(iids_hbm.at[pl.ds(base_row, BPW)], iidx, isem.at[0])
    hu.wait()
    hi.wait()

    def start(coff, p):
        pltpu.async_copy(
            utab_hbm.at[uidx.at[pl.ds(coff, CHUNK)]], ubuf.at[p], usem.at[p]
        )
        pltpu.async_copy(
            itab_hbm.at[iidx.at[pl.ds(coff, CHUNK)]], ibuf.at[p], isem.at[p]
        )

    def wait(p):
        pltpu.make_async_copy(
            utab_hbm.at[uidx.at[pl.ds(0, CHUNK)]], ubuf.at[p], usem.at[p]
        ).wait()
        pltpu.make_async_copy(
            itab_hbm.at[iidx.at[pl.ds(0, CHUNK)]], ibuf.at[p], isem.at[p]
        ).wait()

    lane = lax.iota(jnp.int32, L)
    col_last = jnp.full((L,), L - 1, jnp.int32)

    def compute_chunk(ub, ib, base):
        def group(g, _, ub=ub, ib=ib, base=base):
            SUB = 2
            DEPTH = 1
            pending = []
            for h in range(L // SUB):
                sums_by_row = []
                for j in range(SUB):
                    r = g * L + h * SUB + j
                    prods = [
                        ub[r, pl.ds(k * L, L)] * ib[r, pl.ds(k * L, L)]
                        for k in range(D // L)
                    ]
                    while len(prods) > 1:
                        prods = [
                            prods[i] + prods[i + 1] if i + 1 < len(prods) else prods[i]
                            for i in range(0, len(prods), 2)
                        ]
                    sums_by_row.append(plsc.cumsum(prods[0]))
                pending.append((h * SUB, sums_by_row))
                if len(pending) > DEPTH:
                    off, vals = pending.pop(0)
                    for j in range(SUB):
                        part[off + j, pl.ds(0, L)] = vals[j]
            for off, vals in pending:
                for j in range(SUB):
                    part[off + j, pl.ds(0, L)] = vals[j]
            sums = plsc.load_gather(part, [lane, col_last])
            outv[pl.ds(base + g * L, L)] = sums
            return 0

        lax.fori_loop(0, CHUNK // L, group, 0)

    start(0, 0)
    start(CHUNK, 1)

    def chunk_step(c, _):
        p = lax.rem(c, 2)
        base = c * CHUNK
        wait(p)
        compute_chunk(ubuf.at[p], ibuf.at[p], base)

        @pl.when(c < NCHUNK - 2)
        def _():
            start(base + 2 * CHUNK, p)

        return 0

    lax.fori_loop(0, NCHUNK, chunk_step, 0)

    pltpu.sync_copy(outv, out_hbm.at[pl.ds(base_row, BPW)])


def kernel(user_ids, item_ids, user_table, item_table):
    return _mf_sc_kernel(user_ids, item_ids, user_table, item_table)


# speedup vs baseline: 1.0581x; 1.0581x over previous
import functools

import jax
import jax.numpy as jnp
from jax import lax
from jax.experimental import pallas as pl
from jax.experimental.pallas import tpu as pltpu
from jax.experimental.pallas import tpu_sc as plsc

NC = 2
NS = 16
L = 16
NW = NC * NS

BATCH = 16384
D = 128
BPW = BATCH // NW
CHUNK = 64
NCHUNK = BPW // CHUNK

_mesh = plsc.VectorSubcoreMesh(
    core_axis_name="c", subcore_axis_name="s", num_cores=NC, num_subcores=NS
)


@functools.partial(
    pl.kernel,
    out_type=jax.ShapeDtypeStruct((BATCH,), jnp.float32),
    mesh=_mesh,
    compiler_params=pltpu.CompilerParams(needs_layout_passes=False),
    scratch_types=[
        pltpu.VMEM((BPW,), jnp.int32),
        pltpu.VMEM((BPW,), jnp.int32),
        pltpu.VMEM((2, CHUNK, D), jnp.float32),
        pltpu.VMEM((2, CHUNK, D), jnp.float32),
        pltpu.VMEM((BPW,), jnp.float32),
        pltpu.VMEM((L, L + 1), jnp.float32),
        pltpu.SemaphoreType.DMA((2,)),
        pltpu.SemaphoreType.DMA((2,)),
    ],
)
def _mf_sc_kernel(
    uids_hbm, iids_hbm, utab_hbm, itab_hbm, out_hbm,
    uidx, iidx, ubuf, ibuf, outv, part,
    usem, isem,
):
    wid = lax.axis_index("s") * NC + lax.axis_index("c")
    base_row = wid * BPW
    hu = pltpu.async_copy(uids_hbm.at[pl.ds(base_row, BPW)], uidx, usem.at[0])
    hi = pltpu.async_copy(iids_hbm.at[pl.ds(base_row, BPW)], iidx, isem.at[0])
    hu.wait()
    hi.wait()

    def start(coff, p):
        pltpu.async_copy(
            utab_hbm.at[uidx.at[pl.ds(coff, CHUNK)]], ubuf.at[p], usem.at[p]
        )
        pltpu.async_copy(
            itab_hbm.at[iidx.at[pl.ds(coff, CHUNK)]], ibuf.at[p], isem.at[p]
        )

    def wait(p):
        pltpu.make_async_copy(
            utab_hbm.at[uidx.at[pl.ds(0, CHUNK)]], ubuf.at[p], usem.at[p]
        ).wait()
        pltpu.make_async_copy(
            itab_hbm.at[iidx.at[pl.ds(0, CHUNK)]], ibuf.at[p], isem.at[p]
        ).wait()

    lane = lax.iota(jnp.int32, L)
    col_last = jnp.full((L,), L - 1, jnp.int32)

    def compute_chunk(ub, ib, base):
        def group(g, _, ub=ub, ib=ib, base=base):
            SUB = 2
            DEPTH = 1
            pending = []
            for h in range(L // SUB):
                sums_by_row = []
                for j in range(SUB):
                    r = g * L + h * SUB + j
                    prods = [
                        ub[r, pl.ds(k * L, L)] * ib[r, pl.ds(k * L, L)]
                        for k in range(D // L)
                    ]
                    while len(prods) > 1:
                        prods = [
                            prods[i] + prods[i + 1] if i + 1 < len(prods) else prods[i]
                            for i in range(0, len(prods), 2)
                        ]
                    sums_by_row.append(plsc.cumsum(prods[0]))
                pending.append((h * SUB, sums_by_row))
                if len(pending) > DEPTH:
                    off, vals = pending.pop(0)
                    for j in range(SUB):
                        part[off + j, pl.ds(0, L)] = vals[j]
            for off, vals in pending:
                for j in range(SUB):
                    part[off + j, pl.ds(0, L)] = vals[j]
            sums = plsc.load_gather(part, [lane, col_last])
            outv[pl.ds(base + g * L, L)] = sums
            return 0

        lax.fori_loop(0, CHUNK // L, group, 0)

    start(0, 0)
    start(CHUNK, 1)

    def chunk_step(c, _):
        p = lax.rem(c, 2)
        base = c * CHUNK
        wait(p)
        compute_chunk(ubuf.at[p], ibuf.at[p], base)

        @pl.when(c < NCHUNK - 2)
        def _():
            start(base + 2 * CHUNK, p)

        return 0

    lax.fori_loop(0, NCHUNK, chunk_step, 0)

    pltpu.sync_copy(outv, out_hbm.at[pl.ds(base_row, BPW)])


def kernel(user_ids, item_ids, user_table, item_table):
    return _mf_sc_kernel(user_ids, item_ids, user_table, item_table)
